# baseline (speedup 1.0000x reference)
import jax
import jax.numpy as jnp
from jax import lax
from jax.experimental import pallas as pl
from jax.experimental.pallas import tpu as pltpu
from jax.experimental.pallas import tpu_sc as plsc

B = 16384
OUT_D = 149
NC = 2
NS = 16
NW = NC * NS
ROWS_PER_W = B // NW
CHUNK = 128
NCHUNK = ROWS_PER_W // CHUNK
NGROUP = CHUNK // 16

COL_UNIT = 0
COL_NUM = 64
COL_AB = 75
COL_TR = 91
COL_ST = 107
COL_RES = 123
COL_DEF = 129
COL_MOV = 139

F_AB = 0
F_TR = 4
F_ST = 7
F_MOV = 9
F_NUM = 19
F_RES = 30
F_W = 46

OUT_W = CHUNK * OUT_D


def _full(v):
    return jnp.full((16,), v, jnp.int32)


def _prep_exp_table(tab_v, q_v, e_v):
    lanes = lax.iota(jnp.int32, 16)
    s = jnp.zeros((16,), jnp.float32)
    for d in range(16):
        s = s + (plsc.load_gather(tab_v, [lanes, _full(d)])
                 * plsc.load_gather(q_v, [_full(d)]))
    e_v[...] = s
    s0 = plsc.load_gather(e_v, [_full(0)])
    e_v[...] = jnp.exp(s - s0)


def _body(uids, feat, utab, atab, ttab, stab, qa, qt, qs,
          out,
          uids_v, feat_v, rows_v, out_v,
          atab_v, ttab_v, stab_v, ea_v, et_v, es_v,
          qa_v, qt_v, qs_v, sem_in, sem_g, sem_out):
    wid = lax.axis_index("s") * NC + lax.axis_index("c")
    base_w = wid * ROWS_PER_W

    pltpu.sync_copy(atab, atab_v.at[pl.ds(0, 14)])
    pltpu.sync_copy(ttab, ttab_v.at[pl.ds(0, 12)])
    pltpu.sync_copy(stab, stab_v.at[pl.ds(0, 4)])
    pltpu.sync_copy(qa, qa_v)
    pltpu.sync_copy(qt, qt_v)
    pltpu.sync_copy(qs, qs_v)
    _prep_exp_table(atab_v, qa_v, ea_v)
    _prep_exp_table(ttab_v, qt_v, et_v)
    _prep_exp_table(stab_v, qs_v, es_v)

    def stage(c, b):
        base = base_w + c * CHUNK
        mk = pltpu.async_copy
        return [
            mk(uids.at[pl.ds(base, CHUNK)], uids_v.at[b], sem_in.at[b]),
            mk(feat.at[pl.ds(base, CHUNK)], feat_v.at[b], sem_in.at[b]),
        ]

    def attend(featb, f_col, n_l, tab_v, e_v, out_col, rowid, rowoff, outb):
        idxs = [plsc.bitcast(
            plsc.load_gather(featb, [rowid, _full(f_col + l)]), jnp.int32)
            for l in range(n_l)]
        es = [plsc.load_gather(e_v, [ix]) for ix in idxs]
        denom = es[0]
        for e in es[1:]:
            denom = denom + e
        inv = 1.0 / denom
        ws = [e * inv for e in es]
        for d in range(16):
            acc = ws[0] * plsc.load_gather(tab_v, [idxs[0], _full(d)])
            for l in range(1, n_l):
                acc = acc + ws[l] * plsc.load_gather(tab_v, [idxs[l], _full(d)])
            plsc.store_scatter(outb, [rowoff + _full(out_col + d)], acc)

    in_descs = {0: stage(0, 0)}
    g_descs = {}
    out_descs = {}
    for c in range(NCHUNK):
        b = c % 2
        base = base_w + c * CHUNK
        with jax.named_scope("wait_in"):
            for d in in_descs.pop(c):
                d.wait()
        g_descs[c] = pltpu.async_copy(utab.at[uids_v.at[b]],
                                      rows_v.at[b], sem_g.at[b])
        if c + 1 < NCHUNK:
            in_descs[c + 1] = stage(c + 1, 1 - b)
        if c - 2 >= 0:
            out_descs.pop(c - 2).wait()

        featb = feat_v.at[b]
        outb, rowsb = out_v.at[b], rows_v.at[b]

        def group_ac(g, carry):
            rbase = g * 16
            for j in range(16):
                r = rbase + j
                roff = r * OUT_D
                outb[pl.ds(roff + COL_NUM, 16)] = featb[r, pl.ds(F_NUM, 16)]
                outb[pl.ds(roff + COL_RES, 16)] = featb[r, pl.ds(F_RES, 16)]
                outb[pl.ds(roff + COL_MOV, 16)] = featb[r, pl.ds(F_MOV, 16)] * 0.1
            rowid = lax.iota(jnp.int32, 16) + rbase
            rowoff = rowid * OUT_D
            attend(featb, F_AB, 4, atab_v, ea_v, COL_AB, rowid, rowoff, outb)
            attend(featb, F_TR, 3, ttab_v, et_v, COL_TR, rowid, rowoff, outb)
            attend(featb, F_ST, 2, stab_v, es_v, COL_ST, rowid, rowoff, outb)
            return carry

        with jax.named_scope("phase_ac"):
            lax.fori_loop(0, NGROUP, group_ac, 0)
        with jax.named_scope("wait_gather"):
            g_descs.pop(c).wait()

        def group_b(g, carry):
            rbase = g * 16
            for j in range(16):
                r = rbase + j
                roff = r * OUT_D
                for k in range(4):
                    outb[pl.ds(roff + k * 16, 16)] = rowsb[r, pl.ds(k * 16, 16)]
            return carry

        with jax.named_scope("phase_b"):
            lax.fori_loop(0, NGROUP, group_b, 0)
        out_descs[c] = pltpu.async_copy(
            out_v.at[b, pl.ds(0, OUT_W)],
            out.at[pl.ds(base * OUT_D, OUT_W)], sem_out.at[b])
    for c in sorted(out_descs):
        out_descs.pop(c).wait()


def kernel(unit_type_ids, ability_indices, trait_indices, status_indices,
           numerical, resistances, defenses, movement_costs,
           unit_type_table, ability_table, trait_table, status_table,
           ability_query, trait_query, status_query):
    mesh = plsc.VectorSubcoreMesh(core_axis_name="c", subcore_axis_name="s")
    f32 = jnp.float32
    i32 = jnp.int32
    bc = lambda x: lax.bitcast_convert_type(x.astype(i32), f32)
    feat = jnp.concatenate(
        [bc(ability_indices), bc(trait_indices), bc(status_indices),
         movement_costs, numerical, resistances, defenses],
        axis=1)
    kfn = pl.kernel(
        _body,
        mesh=mesh,
        compiler_params=pltpu.CompilerParams(
            use_tc_tiling_on_sc=False, needs_layout_passes=False),
        out_type=jax.ShapeDtypeStruct((B * OUT_D,), f32),
        scratch_types=[
            pltpu.VMEM((2, CHUNK), i32),
            pltpu.VMEM((2, CHUNK, F_W), f32),
            pltpu.VMEM((2, CHUNK, 64), f32),
            pltpu.VMEM((2, OUT_W + 16), f32),
            pltpu.VMEM((16, 16), f32),
            pltpu.VMEM((16, 16), f32),
            pltpu.VMEM((16, 16), f32),
            pltpu.VMEM((16,), f32),
            pltpu.VMEM((16,), f32),
            pltpu.VMEM((16,), f32),
            pltpu.VMEM((16,), f32),
            pltpu.VMEM((16,), f32),
            pltpu.VMEM((16,), f32),
            pltpu.SemaphoreType.DMA((2,)),
            pltpu.SemaphoreType.DMA((2,)),
            pltpu.SemaphoreType.DMA((2,)),
        ],
    )
    out_flat = kfn(
        unit_type_ids.astype(i32), feat,
        unit_type_table, ability_table, trait_table, status_table,
        ability_query, trait_query, status_query,
    )
    return out_flat.reshape(B, OUT_D)


# speedup vs baseline: 1.0246x; 1.0246x over previous
import jax
import jax.numpy as jnp
from jax import lax
from jax.experimental import pallas as pl
from jax.experimental.pallas import tpu as pltpu
from jax.experimental.pallas import tpu_sc as plsc

B = 16384
OUT_D = 149
NC = 2
NS = 16
NW = NC * NS
ROWS_PER_W = B // NW
CHUNK = 128
NCHUNK = ROWS_PER_W // CHUNK
NGROUP = CHUNK // 16

COL_UNIT = 0
COL_NUM = 64
COL_AB = 75
COL_TR = 91
COL_ST = 107
COL_RES = 123
COL_DEF = 129
COL_MOV = 139

F_AB = 0
F_TR = 4
F_ST = 7
F_MOV = 9
F_NUM = 19
F_RES = 30
F_DEF = 36
F_W = 52

OUT_P = 256
OUT_W = CHUNK * OUT_P


def _full(v):
    return jnp.full((16,), v, jnp.int32)


def _prep_exp_table(tab_v, q_v, e_v):
    lanes = lax.iota(jnp.int32, 16)
    s = jnp.zeros((16,), jnp.float32)
    for d in range(16):
        s = s + (plsc.load_gather(tab_v, [lanes, _full(d)])
                 * plsc.load_gather(q_v, [_full(d)]))
    e_v[...] = s
    s0 = plsc.load_gather(e_v, [_full(0)])
    e_v[...] = jnp.exp(s - s0)


def _body(uids, feat, utab, atab, ttab, stab, qa, qt, qs,
          out,
          uids_v, feat_v, rows_v, out_v,
          atab_v, ttab_v, stab_v, ea_v, et_v, es_v,
          qa_v, qt_v, qs_v, sem_in, sem_g, sem_out):
    wid = lax.axis_index("s") * NC + lax.axis_index("c")
    base_w = wid * ROWS_PER_W

    pltpu.sync_copy(atab, atab_v.at[pl.ds(0, 14)])
    pltpu.sync_copy(ttab, ttab_v.at[pl.ds(0, 12)])
    pltpu.sync_copy(stab, stab_v.at[pl.ds(0, 4)])
    pltpu.sync_copy(qa, qa_v)
    pltpu.sync_copy(qt, qt_v)
    pltpu.sync_copy(qs, qs_v)
    _prep_exp_table(atab_v, qa_v, ea_v)
    _prep_exp_table(ttab_v, qt_v, et_v)
    _prep_exp_table(stab_v, qs_v, es_v)

    def stage(c, b):
        base = base_w + c * CHUNK
        mk = pltpu.async_copy
        return [
            mk(uids.at[pl.ds(base, CHUNK)], uids_v.at[b], sem_in.at[b]),
            mk(feat.at[pl.ds(base, CHUNK)], feat_v.at[b], sem_in.at[b]),
        ]

    def attend(featb, f_col, n_l, tab_v, e_v, out_col, rowid, rowoff, outb):
        idxs = [plsc.bitcast(
            plsc.load_gather(featb, [rowid, _full(f_col + l)]), jnp.int32)
            for l in range(n_l)]
        es = [plsc.load_gather(e_v, [ix]) for ix in idxs]
        denom = es[0]
        for e in es[1:]:
            denom = denom + e
        inv = 1.0 / denom
        ws = [e * inv for e in es]
        for d in range(16):
            acc = ws[0] * plsc.load_gather(tab_v, [idxs[0], _full(d)])
            for l in range(1, n_l):
                acc = acc + ws[l] * plsc.load_gather(tab_v, [idxs[l], _full(d)])
            plsc.store_scatter(outb, [rowoff + _full(out_col + d)], acc)

    in_descs = {0: stage(0, 0)}
    g_descs = {}
    out_descs = {}
    for c in range(NCHUNK):
        b = c % 2
        base = base_w + c * CHUNK
        with jax.named_scope("wait_in"):
            for d in in_descs.pop(c):
                d.wait()
        g_descs[c] = pltpu.async_copy(utab.at[uids_v.at[b]],
                                      rows_v.at[b], sem_g.at[b])
        if c + 1 < NCHUNK:
            in_descs[c + 1] = stage(c + 1, 1 - b)
        if c - 2 >= 0:
            out_descs.pop(c - 2).wait()

        featb = feat_v.at[b]
        outb, rowsb = out_v.at[b], rows_v.at[b]

        def group_ac(g, carry):
            rbase = g * 16
            for j in range(16):
                r = rbase + j
                roff = (r // 8) * 2048 + (r % 8) * 128
                roff1 = roff + 1024 - 128
                outb[pl.ds(roff + COL_NUM, 16)] = featb[r, pl.ds(F_NUM, 16)]
                outb[pl.ds(roff1 + COL_DEF, 16)] = featb[r, pl.ds(F_DEF, 16)]
                outb[pl.ds(roff1 + COL_MOV, 16)] = featb[r, pl.ds(F_MOV, 16)] * 0.1
            rowid = lax.iota(jnp.int32, 16) + rbase
            rowphys = ((rowid >> 3) << 11) + ((rowid & 7) << 7)
            for cc in range(6):
                col = COL_RES + cc
                pcol = col if col < 128 else 1024 + col - 128
                v = plsc.load_gather(featb, [rowid, _full(F_RES + cc)])
                plsc.store_scatter(outb, [rowphys + _full(pcol)], v)
            attend(featb, F_AB, 4, atab_v, ea_v, COL_AB, rowid, rowphys, outb)
            attend(featb, F_TR, 3, ttab_v, et_v, COL_TR, rowid, rowphys, outb)
            attend(featb, F_ST, 2, stab_v, es_v, COL_ST, rowid, rowphys, outb)
            return carry

        with jax.named_scope("phase_ac"):
            lax.fori_loop(0, NGROUP, group_ac, 0)
        with jax.named_scope("wait_gather"):
            g_descs.pop(c).wait()

        def group_b(g, carry):
            rbase = g * 16
            for j in range(16):
                r = rbase + j
                roff = (r // 8) * 2048 + (r % 8) * 128
                for k in range(4):
                    outb[pl.ds(roff + k * 16, 16)] = rowsb[r, pl.ds(k * 16, 16)]
            return carry

        with jax.named_scope("phase_b"):
            lax.fori_loop(0, NGROUP, group_b, 0)
        out_descs[c] = pltpu.async_copy(
            out_v.at[b, pl.ds(0, OUT_W)],
            out.at[pl.ds(base * OUT_P, OUT_W)], sem_out.at[b])
    for c in sorted(out_descs):
        out_descs.pop(c).wait()


def kernel(unit_type_ids, ability_indices, trait_indices, status_indices,
           numerical, resistances, defenses, movement_costs,
           unit_type_table, ability_table, trait_table, status_table,
           ability_query, trait_query, status_query):
    mesh = plsc.VectorSubcoreMesh(core_axis_name="c", subcore_axis_name="s")
    f32 = jnp.float32
    i32 = jnp.int32
    bc = lambda x: lax.bitcast_convert_type(x.astype(i32), f32)
    feat = jnp.concatenate(
        [bc(ability_indices), bc(trait_indices), bc(status_indices),
         movement_costs, numerical, resistances, defenses,
         jnp.zeros((B, F_W - 46), jnp.float32)],
        axis=1)
    kfn = pl.kernel(
        _body,
        mesh=mesh,
        compiler_params=pltpu.CompilerParams(
            use_tc_tiling_on_sc=False, needs_layout_passes=False),
        out_type=jax.ShapeDtypeStruct((B * OUT_P,), f32),
        scratch_types=[
            pltpu.VMEM((2, CHUNK), i32),
            pltpu.VMEM((2, CHUNK, F_W), f32),
            pltpu.VMEM((2, CHUNK, 64), f32),
            pltpu.VMEM((2, OUT_W + 16), f32),
            pltpu.VMEM((16, 16), f32),
            pltpu.VMEM((16, 16), f32),
            pltpu.VMEM((16, 16), f32),
            pltpu.VMEM((16,), f32),
            pltpu.VMEM((16,), f32),
            pltpu.VMEM((16,), f32),
            pltpu.VMEM((16,), f32),
            pltpu.VMEM((16,), f32),
            pltpu.VMEM((16,), f32),
            pltpu.SemaphoreType.DMA((2,)),
            pltpu.SemaphoreType.DMA((2,)),
            pltpu.SemaphoreType.DMA((2,)),
        ],
    )
    out_flat = kfn(
        unit_type_ids.astype(i32), feat,
        unit_type_table, ability_table, trait_table, status_table,
        ability_query, trait_query, status_query,
    )
    out4 = out_flat.reshape(B // 8, 2, 8, 128)
    return out4.swapaxes(1, 2).reshape(B, OUT_P)[:, :OUT_D]


# speedup vs baseline: 1.2045x; 1.1757x over previous
import jax
import jax.numpy as jnp
from jax import lax
from jax.experimental import pallas as pl
from jax.experimental.pallas import tpu as pltpu
from jax.experimental.pallas import tpu_sc as plsc

B = 16384
NC = 2
NS = 16
NW = NC * NS
ROWS_PER_W = B // NW
CHUNK = 128
NCHUNK = ROWS_PER_W // CHUNK
NGROUP = CHUNK // 16

A_W = 85
A_NUM = 0
A_AB = 11
A_TR = 27
A_ST = 43
A_RESDEF = 59
A_MOV = 75

F_AB = 0
F_TR = 4
F_ST = 7
F_MOV = 9
F_NUM = 19
F_RES = 30
F_W = 46


def _full(v):
    return jnp.full((16,), v, jnp.int32)


def _prep_exp_table(tab_v, q_v, e_v):
    lanes = lax.iota(jnp.int32, 16)
    s = jnp.zeros((16,), jnp.float32)
    for d in range(16):
        s = s + (plsc.load_gather(tab_v, [lanes, _full(d)])
                 * plsc.load_gather(q_v, [_full(d)]))
    e_v[...] = s
    s0 = plsc.load_gather(e_v, [_full(0)])
    e_v[...] = jnp.exp(s - s0)


def _gather_body(uids, utab, out,
                 uids_v, rows_v, out_v, sem_in, sem_g, sem_out):
    wid = lax.axis_index("s") * NC + lax.axis_index("c")
    base_w = wid * ROWS_PER_W

    in_descs = {0: pltpu.async_copy(
        uids.at[pl.ds(base_w, CHUNK)], uids_v.at[0], sem_in.at[0])}
    g_descs = {}
    out_descs = {}
    for c in range(NCHUNK):
        b = c % 2
        base = base_w + c * CHUNK
        in_descs.pop(c).wait()
        g_descs[c] = pltpu.async_copy(utab.at[uids_v.at[b]],
                                      rows_v.at[b], sem_g.at[b])
        if c + 1 < NCHUNK:
            in_descs[c + 1] = pltpu.async_copy(
                uids.at[pl.ds(base + CHUNK, CHUNK)], uids_v.at[1 - b],
                sem_in.at[1 - b])
        if c - 2 >= 0:
            out_descs.pop(c - 2).wait()
        g_descs.pop(c).wait()
        rowsb, outb = rows_v.at[b], out_v.at[b]

        def group(g, carry):
            rbase = g * 16
            for j in range(16):
                r = rbase + j
                for k in range(4):
                    outb[pl.ds(r * 64 + k * 16, 16)] = rowsb[r, pl.ds(k * 16, 16)]
            return carry

        lax.fori_loop(0, NGROUP, group, 0)
        out_descs[c] = pltpu.async_copy(
            out_v.at[b], out.at[pl.ds(base * 64, CHUNK * 64)], sem_out.at[b])
    for c in sorted(out_descs):
        out_descs.pop(c).wait()


def _encode_body(feat, atab, ttab, stab, qa, qt, qs,
                 out,
                 feat_v, out_v, atab_v, ttab_v, stab_v, ea_v, et_v, es_v,
                 qa_v, qt_v, qs_v, sem_in, sem_out):
    wid = lax.axis_index("s") * NC + lax.axis_index("c")
    base_w = wid * ROWS_PER_W

    pltpu.sync_copy(atab, atab_v.at[pl.ds(0, 14)])
    pltpu.sync_copy(ttab, ttab_v.at[pl.ds(0, 12)])
    pltpu.sync_copy(stab, stab_v.at[pl.ds(0, 4)])
    pltpu.sync_copy(qa, qa_v)
    pltpu.sync_copy(qt, qt_v)
    pltpu.sync_copy(qs, qs_v)
    _prep_exp_table(atab_v, qa_v, ea_v)
    _prep_exp_table(ttab_v, qt_v, et_v)
    _prep_exp_table(stab_v, qs_v, es_v)

    def attend(featb, f_col, n_l, tab_v, e_v, out_col, rowid, rowoff, outb):
        idxs = [plsc.bitcast(
            plsc.load_gather(featb, [rowid, _full(f_col + l)]), jnp.int32)
            for l in range(n_l)]
        es = [plsc.load_gather(e_v, [ix]) for ix in idxs]
        denom = es[0]
        for e in es[1:]:
            denom = denom + e
        inv = 1.0 / denom
        ws = [e * inv for e in es]
        for d in range(16):
            acc = ws[0] * plsc.load_gather(tab_v, [idxs[0], _full(d)])
            for l in range(1, n_l):
                acc = acc + ws[l] * plsc.load_gather(tab_v, [idxs[l], _full(d)])
            plsc.store_scatter(outb, [rowoff + _full(out_col + d)], acc)

    in_descs = {0: pltpu.async_copy(
        feat.at[pl.ds(base_w, CHUNK)], feat_v.at[0], sem_in.at[0])}
    out_descs = {}
    for c in range(NCHUNK):
        b = c % 2
        base = base_w + c * CHUNK
        in_descs.pop(c).wait()
        if c + 1 < NCHUNK:
            in_descs[c + 1] = pltpu.async_copy(
                feat.at[pl.ds(base + CHUNK, CHUNK)], feat_v.at[1 - b],
                sem_in.at[1 - b])
        if c - 2 >= 0:
            out_descs.pop(c - 2).wait()
        featb, outb = feat_v.at[b], out_v.at[b]

        def group_ac(g, carry):
            rbase = g * 16
            for j in range(16):
                r = rbase + j
                roff = r * A_W
                outb[pl.ds(roff + A_NUM, 16)] = featb[r, pl.ds(F_NUM, 16)]
                outb[pl.ds(roff + A_RESDEF, 16)] = featb[r, pl.ds(F_RES, 16)]
                outb[pl.ds(roff + A_MOV, 16)] = featb[r, pl.ds(F_MOV, 16)] * 0.1
            rowid = lax.iota(jnp.int32, 16) + rbase
            rowoff = rowid * A_W
            attend(featb, F_AB, 4, atab_v, ea_v, A_AB, rowid, rowoff, outb)
            attend(featb, F_TR, 3, ttab_v, et_v, A_TR, rowid, rowoff, outb)
            attend(featb, F_ST, 2, stab_v, es_v, A_ST, rowid, rowoff, outb)
            return carry

        lax.fori_loop(0, NGROUP, group_ac, 0)
        out_descs[c] = pltpu.async_copy(
            out_v.at[b, pl.ds(0, CHUNK * A_W)],
            out.at[pl.ds(base * A_W, CHUNK * A_W)], sem_out.at[b])
    for c in sorted(out_descs):
        out_descs.pop(c).wait()


def kernel(unit_type_ids, ability_indices, trait_indices, status_indices,
           numerical, resistances, defenses, movement_costs,
           unit_type_table, ability_table, trait_table, status_table,
           ability_query, trait_query, status_query):
    mesh = plsc.VectorSubcoreMesh(core_axis_name="c", subcore_axis_name="s")
    f32 = jnp.float32
    i32 = jnp.int32
    cp = pltpu.CompilerParams(
        use_tc_tiling_on_sc=False, needs_layout_passes=False)
    bc = lambda x: lax.bitcast_convert_type(x.astype(i32), f32)
    feat = jnp.concatenate(
        [bc(ability_indices), bc(trait_indices), bc(status_indices),
         movement_costs, numerical, resistances, defenses],
        axis=1)

    gather_k = pl.kernel(
        _gather_body,
        mesh=mesh,
        compiler_params=cp,
        out_type=jax.ShapeDtypeStruct((B * 64,), f32),
        scratch_types=[
            pltpu.VMEM((2, CHUNK), i32),
            pltpu.VMEM((2, CHUNK, 64), f32),
            pltpu.VMEM((2, CHUNK * 64), f32),
            pltpu.SemaphoreType.DMA((2,)),
            pltpu.SemaphoreType.DMA((2,)),
            pltpu.SemaphoreType.DMA((2,)),
        ],
    )
    encode_k = pl.kernel(
        _encode_body,
        mesh=mesh,
        compiler_params=cp,
        out_type=jax.ShapeDtypeStruct((B * A_W,), f32),
        scratch_types=[
            pltpu.VMEM((2, CHUNK, F_W), f32),
            pltpu.VMEM((2, CHUNK * A_W + 16), f32),
            pltpu.VMEM((16, 16), f32),
            pltpu.VMEM((16, 16), f32),
            pltpu.VMEM((16, 16), f32),
            pltpu.VMEM((16,), f32),
            pltpu.VMEM((16,), f32),
            pltpu.VMEM((16,), f32),
            pltpu.VMEM((16,), f32),
            pltpu.VMEM((16,), f32),
            pltpu.VMEM((16,), f32),
            pltpu.SemaphoreType.DMA((2,)),
            pltpu.SemaphoreType.DMA((2,)),
        ],
    )
    unit_flat = gather_k(unit_type_ids.astype(i32), unit_type_table)
    enc_flat = encode_k(feat, ability_table, trait_table, status_table,
                        ability_query, trait_query, status_query)
    return jnp.concatenate(
        [unit_flat.reshape(B, 64), enc_flat.reshape(B, A_W)], axis=1)
